# baseline (speedup 1.0000x reference)
import functools
import jax
import jax.numpy as jnp
from jax import lax
from jax.experimental import pallas as pl
from jax.experimental.pallas import tpu as pltpu
from jax.experimental.pallas import tpu_sc as plsc

N = 10000
NP = 10240
E = 320000
D = 128
NF = 256
CUTOFF = 10.0
_LOG2 = 0.6931471805599453

NC = 2
NS = 16
NW = NC * NS

_MESH = functools.partial(
    plsc.VectorSubcoreMesh, core_axis_name="c", subcore_axis_name="s")


def _ssp_tc(v):
    return jnp.maximum(v, 0.0) + jnp.log1p(jnp.exp(-jnp.abs(v))) - _LOG2


def _gather_loop(t_hbm, idx_hbm, out_hbm, idx_v, rows_v, isem, gsem, osem,
                 base0, out0, ncols, NB, CH, ITER):
    for b in range(NB):
        pltpu.async_copy(idx_hbm.at[pl.ds(base0 + b * CH, CH)],
                         idx_v.at[b], isem)

    def outer(g, _):
        off0 = out0 + g * (NB * CH)

        @pl.when(g > 0)
        def _drain_writes():
            for b in range(NB):
                pltpu.make_async_copy(
                    rows_v.at[b], out_hbm.at[pl.ds(out0, CH), :],
                    osem).wait()

        for b in range(NB):
            pltpu.make_async_copy(
                idx_hbm.at[pl.ds(base0, CH)], idx_v.at[b], isem).wait()
            pltpu.async_copy(t_hbm.at[idx_v.at[b]], rows_v.at[b], gsem)

        for b in range(NB):
            off = off0 + b * CH
            pltpu.make_async_copy(
                t_hbm.at[idx_v.at[b]], rows_v.at[b], gsem).wait()
            pltpu.async_copy(rows_v.at[b],
                             out_hbm.at[pl.ds(off, CH), :], osem)

            @pl.when(g + 1 < ITER)
            def _refill():
                noff = base0 + (g + 1) * (NB * CH) + b * CH
                pltpu.async_copy(idx_hbm.at[pl.ds(noff, CH)],
                                 idx_v.at[b], isem)
        return _

    lax.fori_loop(0, ITER, outer, None)
    for b in range(NB):
        pltpu.make_async_copy(
            rows_v.at[b], out_hbm.at[pl.ds(out0, CH), :], osem).wait()


def _sc_gather_rows(table, idx, ncols):
    EW = E // NW
    NB = 5
    CH = 40 if ncols > 256 else 80
    ITER = EW // (CH * NB)

    @functools.partial(
        pl.kernel,
        out_type=jax.ShapeDtypeStruct((E, ncols), jnp.float32),
        mesh=_MESH(),
        scratch_types=[
            pltpu.VMEM((NB, CH), jnp.int32),
            pltpu.VMEM((NB, CH, ncols), jnp.float32),
            pltpu.SemaphoreType.DMA,
            pltpu.SemaphoreType.DMA,
            pltpu.SemaphoreType.DMA,
        ],
    )
    def k(t_hbm, idx_hbm, out_hbm, idx_v, rows_v, isem, gsem, osem):
        wid = lax.axis_index("s") * NC + lax.axis_index("c")
        base0 = wid * EW
        _gather_loop(t_hbm, idx_hbm, out_hbm, idx_v, rows_v,
                     isem, gsem, osem, base0, base0, ncols, NB, CH, ITER)

    return k(table, idx)


def _sc_gather_src_dst(ht, xp, src, dst):
    EW = E // NW
    NB = 5
    CHS, CHD = 40, 80
    ITS = EW // (CHS * NB)
    ITD = EW // (CHD * NB)
    W = NF + 128

    @functools.partial(
        pl.kernel,
        out_type=(jax.ShapeDtypeStruct((E, W), jnp.float32),
                  jax.ShapeDtypeStruct((E, 128), jnp.float32)),
        mesh=_MESH(),
        scratch_types=[
            pltpu.VMEM((NB, CHS), jnp.int32),
            pltpu.VMEM((NB, CHS, W), jnp.float32),
            pltpu.VMEM((NB, CHD), jnp.int32),
            pltpu.VMEM((NB, CHD, 128), jnp.float32),
            pltpu.SemaphoreType.DMA,
            pltpu.SemaphoreType.DMA,
            pltpu.SemaphoreType.DMA,
        ],
    )
    def k(ht_hbm, xp_hbm, src_hbm, dst_hbm, gs_hbm, xpd_hbm,
          idx_s, rows_s, idx_d, rows_d, isem, gsem, osem):
        wid = lax.axis_index("s") * NC + lax.axis_index("c")
        base0 = wid * EW
        _gather_loop(ht_hbm, src_hbm, gs_hbm, idx_s, rows_s,
                     isem, gsem, osem, base0, base0, W, NB, CHS, ITS)
        _gather_loop(xp_hbm, dst_hbm, xpd_hbm, idx_d, rows_d,
                     isem, gsem, osem, base0, base0, 128, NB, CHD, ITD)

    return k(ht, xp, src, dst)


def _sc_gather_xn2(xn, src, dst):
    EW = E // NW
    NB = 5
    CH = 80
    ITER = EW // (CH * NB)

    @functools.partial(
        pl.kernel,
        out_type=jax.ShapeDtypeStruct((2 * E, D), jnp.float32),
        mesh=_MESH(),
        scratch_types=[
            pltpu.VMEM((NB, CH), jnp.int32),
            pltpu.VMEM((NB, CH, D), jnp.float32),
            pltpu.SemaphoreType.DMA,
            pltpu.SemaphoreType.DMA,
            pltpu.SemaphoreType.DMA,
        ],
    )
    def k(t_hbm, src_hbm, dst_hbm, out_hbm, idx_v, rows_v, isem, gsem, osem):
        wid = lax.axis_index("s") * NC + lax.axis_index("c")
        base0 = wid * EW
        _gather_loop(t_hbm, src_hbm, out_hbm, idx_v, rows_v,
                     isem, gsem, osem, base0, base0, D, NB, CH, ITER)
        _gather_loop(t_hbm, dst_hbm, out_hbm, idx_v, rows_v,
                     isem, gsem, osem, base0, E + base0, D, NB, CH, ITER)

    return k(xn, src, dst)


def _sc_segsum(msg, dst3, zeros_nd):
    ET = E // NW
    CH = 40
    NB = 5
    ITER = ET // (CH * NB)
    NCHUNK = ET // CH
    RT = NP // NS

    @functools.partial(
        pl.kernel,
        out_type=jax.ShapeDtypeStruct((2 * NP, D), jnp.float32),
        mesh=_MESH(),
        scratch_types=[
            pltpu.VMEM((NB, CH), jnp.int32),
            pltpu.VMEM((NB, CH, D), jnp.float32),
            pltpu.VMEM_SHARED((NP, D), jnp.float32),
            pltpu.SemaphoreType.DMA,
            pltpu.SemaphoreType.DMA,
        ],
    )
    def k(msg_hbm, dst_hbm, z_hbm, out_hbm, idx_v, rows_v, acc_sh, isem,
          rsem):
        c = lax.axis_index("c")
        t = lax.axis_index("s")
        w = c * NS + t
        base0 = w * ET
        row0 = w * NCHUNK
        pltpu.sync_copy(z_hbm.at[pl.ds(t * RT, RT), :],
                        acc_sh.at[pl.ds(t * RT, RT), :])
        for b in range(NB):
            pltpu.async_copy(dst_hbm.at[row0 + b], idx_v.at[b], isem)
            pltpu.async_copy(msg_hbm.at[pl.ds(base0 + b * CH, CH), :],
                             rows_v.at[b], rsem)
        plsc.subcore_barrier()

        def outer(g, _):
            for b in range(NB):
                j = g * NB + b
                pltpu.make_async_copy(dst_hbm.at[row0], idx_v.at[b],
                                      isem).wait()
                pltpu.make_async_copy(
                    msg_hbm.at[pl.ds(base0, CH), :], rows_v.at[b],
                    rsem).wait()
                pltpu.sync_copy(rows_v.at[b], acc_sh.at[idx_v.at[b]],
                                add=True)

                @pl.when(g + 1 < ITER)
                def _refill():
                    noff = base0 + (j + NB) * CH
                    pltpu.async_copy(dst_hbm.at[row0 + j + NB], idx_v.at[b],
                                     isem)
                    pltpu.async_copy(msg_hbm.at[pl.ds(noff, CH), :],
                                     rows_v.at[b], rsem)
            return _

        lax.fori_loop(0, ITER, outer, None)
        plsc.subcore_barrier()
        pltpu.sync_copy(acc_sh.at[pl.ds(t * RT, RT), :],
                        out_hbm.at[pl.ds(c * NP + t * RT, RT), :])

    return k(msg, dst3, zeros_nd)


def _tc_h(x, W_lin1):
    BN = 2000

    def body(x_ref, w_ref, o_ref):
        o_ref[...] = jnp.dot(x_ref[...], w_ref[...],
                             preferred_element_type=jnp.float32)

    return pl.pallas_call(
        body,
        grid=(N // BN,),
        in_specs=[
            pl.BlockSpec((BN, D), lambda i: (i, 0)),
            pl.BlockSpec((D, NF), lambda i: (0, 0)),
        ],
        out_specs=pl.BlockSpec((BN, NF), lambda i: (i, 0)),
        out_shape=jax.ShapeDtypeStruct((N, NF), jnp.float32),
    )(x, W_lin1)


def _tc_msg(ea, gs, xpd, W1, b1, W2, b2, WL2):
    BE = 2000

    bf = jnp.bfloat16

    def body(ea_ref, gs_ref, xd_ref, w1_ref, b1_ref, w2_ref, b2_ref, wl2_ref,
             o_ref):
        u = _ssp_tc(jnp.dot(ea_ref[...].astype(bf), w1_ref[...].astype(bf),
                            preferred_element_type=jnp.float32) + b1_ref[...])
        wf = jnp.dot(u.astype(bf), w2_ref[...].astype(bf),
                     preferred_element_type=jnp.float32) + b2_ref[...]
        hs = gs_ref[:, :NF]
        xs = gs_ref[:, NF:]
        df = xs - xd_ref[...]
        d2 = jnp.sum(df * df, axis=1, keepdims=True)
        dist = jnp.sqrt(d2 + 1e-12)
        cc = 0.5 * (jnp.cos(dist * (jnp.pi / CUTOFF)) + 1.0)
        cc = jnp.where(dist < CUTOFF, cc, 0.0)
        msg = wf * cc * hs
        o_ref[...] = jnp.dot(msg.astype(bf), wl2_ref[...].astype(bf),
                             preferred_element_type=jnp.float32)

    grid = (E // BE,)
    return pl.pallas_call(
        body,
        grid=grid,
        in_specs=[
            pl.BlockSpec((BE, D), lambda i: (i, 0)),
            pl.BlockSpec((BE, NF + 128), lambda i: (i, 0)),
            pl.BlockSpec((BE, 128), lambda i: (i, 0)),
            pl.BlockSpec((D, NF), lambda i: (0, 0)),
            pl.BlockSpec((1, NF), lambda i: (0, 0)),
            pl.BlockSpec((NF, NF), lambda i: (0, 0)),
            pl.BlockSpec((1, NF), lambda i: (0, 0)),
            pl.BlockSpec((NF, D), lambda i: (0, 0)),
        ],
        out_specs=pl.BlockSpec((BE, D), lambda i: (i, 0)),
        out_shape=jax.ShapeDtypeStruct((E, D), jnp.float32),
    )(ea, gs, xpd, W1, b1, W2, b2, WL2)


def _tc_node(a0, a1, x, b2, W3, b3):
    BN = 2000

    def body(a0_ref, a1_ref, x_ref, b2_ref, w3_ref, b3_ref, o_ref):
        o = _ssp_tc(a0_ref[...] + a1_ref[...] + b2_ref[...])
        o = jnp.dot(o, w3_ref[...],
                    preferred_element_type=jnp.float32) + b3_ref[...]
        o_ref[...] = jnp.maximum(o, 0.0) + x_ref[...]

    return pl.pallas_call(
        body,
        grid=(N // BN,),
        in_specs=[
            pl.BlockSpec((BN, D), lambda i: (i, 0)),
            pl.BlockSpec((BN, D), lambda i: (i, 0)),
            pl.BlockSpec((BN, D), lambda i: (i, 0)),
            pl.BlockSpec((1, D), lambda i: (0, 0)),
            pl.BlockSpec((D, D), lambda i: (0, 0)),
            pl.BlockSpec((1, D), lambda i: (0, 0)),
        ],
        out_specs=pl.BlockSpec((BN, D), lambda i: (i, 0)),
        out_shape=jax.ShapeDtypeStruct((N, D), jnp.float32),
    )(a0, a1, x, b2.reshape(1, D), W3, b3.reshape(1, D))


def _tc_edge(ea, ab, We1, We2, be):
    BE = 2000
    NBLK = E // BE

    bf = jnp.bfloat16

    def body(ea_ref, a_ref, b_ref, w1_ref, w2_ref, bb_ref, o_ref):
        s = a_ref[...] + b_ref[...]
        v = (jnp.dot(ea_ref[...].astype(bf), w1_ref[...].astype(bf),
                     preferred_element_type=jnp.float32)
             + jnp.dot(s.astype(bf), w2_ref[...].astype(bf),
                       preferred_element_type=jnp.float32)
             + bb_ref[...])
        o_ref[...] = jnp.tanh(v) + ea_ref[...]

    return pl.pallas_call(
        body,
        grid=(NBLK,),
        in_specs=[
            pl.BlockSpec((BE, D), lambda i: (i, 0)),
            pl.BlockSpec((BE, D), lambda i: (i, 0)),
            pl.BlockSpec((BE, D), lambda i: (i + NBLK, 0)),
            pl.BlockSpec((D, D), lambda i: (0, 0)),
            pl.BlockSpec((D, D), lambda i: (0, 0)),
            pl.BlockSpec((1, D), lambda i: (0, 0)),
        ],
        out_specs=pl.BlockSpec((BE, D), lambda i: (i, 0)),
        out_shape=jax.ShapeDtypeStruct((E, D), jnp.float32),
    )(ea, ab, ab, We1, We2, be)


def kernel(x, edge_index, edge_attr, x_pos,
           W_mlp1, b_mlp1, W_mlp2, b_mlp2,
           W_lin1, W_lin2, b_lin2, W_lin3, b_lin3,
           W_e, b_e):
    src = edge_index[0]
    dst = edge_index[1]
    xp128 = jnp.pad(x_pos, ((0, 0), (0, 125)))
    h = _tc_h(x, W_lin1)
    ht = jnp.concatenate([h, xp128], axis=1)
    gs, xpd = _sc_gather_src_dst(ht, xp128, src, dst)
    msg = _tc_msg(edge_attr, gs, xpd,
                  W_mlp1, b_mlp1.reshape(1, NF), W_mlp2, b_mlp2.reshape(1, NF),
                  W_lin2)
    zeros_nd = jnp.zeros((NP, D), jnp.float32)
    dst3 = dst.reshape(E // 40, 40)
    parts = _sc_segsum(msg, dst3, zeros_nd)
    xn = _tc_node(parts[:N], parts[NP:NP + N], x, b_lin2, W_lin3, b_lin3)
    ab = _sc_gather_xn2(xn, src, dst)
    edge_out = _tc_edge(edge_attr, ab, W_e[:D], W_e[D:], b_e.reshape(1, D))
    return (xn, edge_out)


# speedup vs baseline: 1.0138x; 1.0138x over previous
import functools
import jax
import jax.numpy as jnp
from jax import lax
from jax.experimental import pallas as pl
from jax.experimental.pallas import tpu as pltpu
from jax.experimental.pallas import tpu_sc as plsc

N = 10000
NP = 10240
E = 320000
D = 128
NF = 256
CUTOFF = 10.0
_LOG2 = 0.6931471805599453

NC = 2
NS = 16
NW = NC * NS

_MESH = functools.partial(
    plsc.VectorSubcoreMesh, core_axis_name="c", subcore_axis_name="s")


def _ssp_tc(v):
    return jnp.maximum(v, 0.0) + jnp.log1p(jnp.exp(-jnp.abs(v))) - _LOG2


def _gather_loop(t_hbm, idx_hbm, out_hbm, idx_v, rows_v, isem, gsem, osem,
                 base0, out0, ncols, NB, CH, ITER):
    for b in range(NB):
        pltpu.async_copy(idx_hbm.at[pl.ds(base0 + b * CH, CH)],
                         idx_v.at[b], isem)

    def outer(g, _):
        off0 = out0 + g * (NB * CH)

        @pl.when(g > 0)
        def _drain_writes():
            for b in range(NB):
                pltpu.make_async_copy(
                    rows_v.at[b], out_hbm.at[pl.ds(out0, CH), :],
                    osem).wait()

        for b in range(NB):
            pltpu.make_async_copy(
                idx_hbm.at[pl.ds(base0, CH)], idx_v.at[b], isem).wait()
            pltpu.async_copy(t_hbm.at[idx_v.at[b]], rows_v.at[b], gsem)

        for b in range(NB):
            off = off0 + b * CH
            pltpu.make_async_copy(
                t_hbm.at[idx_v.at[b]], rows_v.at[b], gsem).wait()
            pltpu.async_copy(rows_v.at[b],
                             out_hbm.at[pl.ds(off, CH), :], osem)

            @pl.when(g + 1 < ITER)
            def _refill():
                noff = base0 + (g + 1) * (NB * CH) + b * CH
                pltpu.async_copy(idx_hbm.at[pl.ds(noff, CH)],
                                 idx_v.at[b], isem)
        return _

    lax.fori_loop(0, ITER, outer, None)
    for b in range(NB):
        pltpu.make_async_copy(
            rows_v.at[b], out_hbm.at[pl.ds(out0, CH), :], osem).wait()


def _sc_gather_rows(table, idx, ncols):
    EW = E // NW
    NB = 5
    CH = 40 if ncols > 256 else 80
    ITER = EW // (CH * NB)

    @functools.partial(
        pl.kernel,
        out_type=jax.ShapeDtypeStruct((E, ncols), jnp.float32),
        mesh=_MESH(),
        scratch_types=[
            pltpu.VMEM((NB, CH), jnp.int32),
            pltpu.VMEM((NB, CH, ncols), jnp.float32),
            pltpu.SemaphoreType.DMA,
            pltpu.SemaphoreType.DMA,
            pltpu.SemaphoreType.DMA,
        ],
    )
    def k(t_hbm, idx_hbm, out_hbm, idx_v, rows_v, isem, gsem, osem):
        wid = lax.axis_index("s") * NC + lax.axis_index("c")
        base0 = wid * EW
        _gather_loop(t_hbm, idx_hbm, out_hbm, idx_v, rows_v,
                     isem, gsem, osem, base0, base0, ncols, NB, CH, ITER)

    return k(table, idx)


def _sc_gather_src_dst(ht, xp, src, dst):
    EW = E // NW
    NB = 5
    CHS, CHD = 40, 80
    ITS = EW // (CHS * NB)
    ITD = EW // (CHD * NB)
    W = NF + 128

    @functools.partial(
        pl.kernel,
        out_type=(jax.ShapeDtypeStruct((E, W), jnp.float32),
                  jax.ShapeDtypeStruct((E, 128), jnp.float32)),
        mesh=_MESH(),
        scratch_types=[
            pltpu.VMEM((NB, CHS), jnp.int32),
            pltpu.VMEM((NB, CHS, W), jnp.float32),
            pltpu.VMEM((NB, CHD), jnp.int32),
            pltpu.VMEM((NB, CHD, 128), jnp.float32),
            pltpu.SemaphoreType.DMA,
            pltpu.SemaphoreType.DMA,
            pltpu.SemaphoreType.DMA,
        ],
    )
    def k(ht_hbm, xp_hbm, src_hbm, dst_hbm, gs_hbm, xpd_hbm,
          idx_s, rows_s, idx_d, rows_d, isem, gsem, osem):
        wid = lax.axis_index("s") * NC + lax.axis_index("c")
        base0 = wid * EW
        _gather_loop(ht_hbm, src_hbm, gs_hbm, idx_s, rows_s,
                     isem, gsem, osem, base0, base0, W, NB, CHS, ITS)
        _gather_loop(xp_hbm, dst_hbm, xpd_hbm, idx_d, rows_d,
                     isem, gsem, osem, base0, base0, 128, NB, CHD, ITD)

    return k(ht, xp, src, dst)


def _sc_gather_xn2(xn, src, dst):
    NE = src.shape[0]
    EW = NE // NW
    NB = 5
    CH = 40
    ITER = EW // (CH * NB)

    @functools.partial(
        pl.kernel,
        out_type=jax.ShapeDtypeStruct((2 * NE, D), jnp.float32),
        mesh=_MESH(),
        scratch_types=[
            pltpu.VMEM((NB, CH), jnp.int32),
            pltpu.VMEM((NB, CH, D), jnp.float32),
            pltpu.SemaphoreType.DMA,
            pltpu.SemaphoreType.DMA,
            pltpu.SemaphoreType.DMA,
        ],
    )
    def k(t_hbm, src_hbm, dst_hbm, out_hbm, idx_v, rows_v, isem, gsem, osem):
        wid = lax.axis_index("s") * NC + lax.axis_index("c")
        base0 = wid * EW
        _gather_loop(t_hbm, src_hbm, out_hbm, idx_v, rows_v,
                     isem, gsem, osem, base0, base0, D, NB, CH, ITER)
        _gather_loop(t_hbm, dst_hbm, out_hbm, idx_v, rows_v,
                     isem, gsem, osem, base0, NE + base0, D, NB, CH, ITER)

    return k(xn, src, dst)


def _sc_segsum(msg, dst3, zeros_nd):
    ET = E // NW
    CH = 40
    NB = 5
    ITER = ET // (CH * NB)
    NCHUNK = ET // CH
    RT = NP // NS

    @functools.partial(
        pl.kernel,
        out_type=jax.ShapeDtypeStruct((2 * NP, D), jnp.float32),
        mesh=_MESH(),
        scratch_types=[
            pltpu.VMEM((NB, CH), jnp.int32),
            pltpu.VMEM((NB, CH, D), jnp.float32),
            pltpu.VMEM_SHARED((NP, D), jnp.float32),
            pltpu.SemaphoreType.DMA,
            pltpu.SemaphoreType.DMA,
        ],
    )
    def k(msg_hbm, dst_hbm, z_hbm, out_hbm, idx_v, rows_v, acc_sh, isem,
          rsem):
        c = lax.axis_index("c")
        t = lax.axis_index("s")
        w = c * NS + t
        base0 = w * ET
        row0 = w * NCHUNK
        pltpu.sync_copy(z_hbm.at[pl.ds(t * RT, RT), :],
                        acc_sh.at[pl.ds(t * RT, RT), :])
        for b in range(NB):
            pltpu.async_copy(dst_hbm.at[row0 + b], idx_v.at[b], isem)
            pltpu.async_copy(msg_hbm.at[pl.ds(base0 + b * CH, CH), :],
                             rows_v.at[b], rsem)
        plsc.subcore_barrier()

        def outer(g, _):
            for b in range(NB):
                j = g * NB + b
                pltpu.make_async_copy(dst_hbm.at[row0], idx_v.at[b],
                                      isem).wait()
                pltpu.make_async_copy(
                    msg_hbm.at[pl.ds(base0, CH), :], rows_v.at[b],
                    rsem).wait()
                pltpu.sync_copy(rows_v.at[b], acc_sh.at[idx_v.at[b]],
                                add=True)

                @pl.when(g + 1 < ITER)
                def _refill():
                    noff = base0 + (j + NB) * CH
                    pltpu.async_copy(dst_hbm.at[row0 + j + NB], idx_v.at[b],
                                     isem)
                    pltpu.async_copy(msg_hbm.at[pl.ds(noff, CH), :],
                                     rows_v.at[b], rsem)
            return _

        lax.fori_loop(0, ITER, outer, None)
        plsc.subcore_barrier()
        pltpu.sync_copy(acc_sh.at[pl.ds(t * RT, RT), :],
                        out_hbm.at[pl.ds(c * NP + t * RT, RT), :])

    return k(msg, dst3, zeros_nd)


def _tc_h(x, W_lin1):
    BN = 2000

    def body(x_ref, w_ref, o_ref):
        o_ref[...] = jnp.dot(x_ref[...], w_ref[...],
                             preferred_element_type=jnp.float32)

    return pl.pallas_call(
        body,
        grid=(N // BN,),
        in_specs=[
            pl.BlockSpec((BN, D), lambda i: (i, 0)),
            pl.BlockSpec((D, NF), lambda i: (0, 0)),
        ],
        out_specs=pl.BlockSpec((BN, NF), lambda i: (i, 0)),
        out_shape=jax.ShapeDtypeStruct((N, NF), jnp.float32),
    )(x, W_lin1)


def _tc_msg(ea, gs, xpd, W1, b1, W2, b2, WL2):
    BE = 2000

    def body(ea_ref, gs_ref, xd_ref, w1_ref, b1_ref, w2_ref, b2_ref, wl2_ref,
             o_ref):
        u = _ssp_tc(jnp.dot(ea_ref[...], w1_ref[...],
                            preferred_element_type=jnp.float32) + b1_ref[...])
        wf = jnp.dot(u, w2_ref[...],
                     preferred_element_type=jnp.float32) + b2_ref[...]
        hs = gs_ref[:, :NF]
        xs = gs_ref[:, NF:]
        df = xs - xd_ref[...]
        d2 = jnp.sum(df * df, axis=1, keepdims=True)
        dist = jnp.sqrt(d2 + 1e-12)
        cc = 0.5 * (jnp.cos(dist * (jnp.pi / CUTOFF)) + 1.0)
        cc = jnp.where(dist < CUTOFF, cc, 0.0)
        msg = wf * cc * hs
        o_ref[...] = jnp.dot(msg, wl2_ref[...],
                             preferred_element_type=jnp.float32)

    grid = (E // BE,)
    return pl.pallas_call(
        body,
        grid=grid,
        in_specs=[
            pl.BlockSpec((BE, D), lambda i: (i, 0)),
            pl.BlockSpec((BE, NF + 128), lambda i: (i, 0)),
            pl.BlockSpec((BE, 128), lambda i: (i, 0)),
            pl.BlockSpec((D, NF), lambda i: (0, 0)),
            pl.BlockSpec((1, NF), lambda i: (0, 0)),
            pl.BlockSpec((NF, NF), lambda i: (0, 0)),
            pl.BlockSpec((1, NF), lambda i: (0, 0)),
            pl.BlockSpec((NF, D), lambda i: (0, 0)),
        ],
        out_specs=pl.BlockSpec((BE, D), lambda i: (i, 0)),
        out_shape=jax.ShapeDtypeStruct((E, D), jnp.float32),
    )(ea, gs, xpd, W1, b1, W2, b2, WL2)


def _tc_node(a0, a1, x, b2, W3, b3):
    BN = 2000

    def body(a0_ref, a1_ref, x_ref, b2_ref, w3_ref, b3_ref, o_ref):
        o = _ssp_tc(a0_ref[...] + a1_ref[...] + b2_ref[...])
        o = jnp.dot(o, w3_ref[...],
                    preferred_element_type=jnp.float32) + b3_ref[...]
        o_ref[...] = jnp.maximum(o, 0.0) + x_ref[...]

    return pl.pallas_call(
        body,
        grid=(N // BN,),
        in_specs=[
            pl.BlockSpec((BN, D), lambda i: (i, 0)),
            pl.BlockSpec((BN, D), lambda i: (i, 0)),
            pl.BlockSpec((BN, D), lambda i: (i, 0)),
            pl.BlockSpec((1, D), lambda i: (0, 0)),
            pl.BlockSpec((D, D), lambda i: (0, 0)),
            pl.BlockSpec((1, D), lambda i: (0, 0)),
        ],
        out_specs=pl.BlockSpec((BN, D), lambda i: (i, 0)),
        out_shape=jax.ShapeDtypeStruct((N, D), jnp.float32),
    )(a0, a1, x, b2.reshape(1, D), W3, b3.reshape(1, D))


def _tc_edge_half(ea, ab, We1, We2, be, b0, prev=None):
    BE = 2000
    NBLK = (ab.shape[0] // 2) // BE

    def body(ea_ref, a_ref, b_ref, w1_ref, w2_ref, bb_ref, *rest):
        o_ref = rest[-1]
        s = a_ref[...] + b_ref[...]
        v = (jnp.dot(ea_ref[...], w1_ref[...],
                     preferred_element_type=jnp.float32)
             + jnp.dot(s, w2_ref[...],
                       preferred_element_type=jnp.float32)
             + bb_ref[...])
        o_ref[...] = jnp.tanh(v) + ea_ref[...]

    ins = [ea, ab, ab, We1, We2, be]
    specs = [
        pl.BlockSpec((BE, D), lambda i: (i + b0, 0)),
        pl.BlockSpec((BE, D), lambda i: (i, 0)),
        pl.BlockSpec((BE, D), lambda i: (i + NBLK, 0)),
        pl.BlockSpec((D, D), lambda i: (0, 0)),
        pl.BlockSpec((D, D), lambda i: (0, 0)),
        pl.BlockSpec((1, D), lambda i: (0, 0)),
    ]
    aliases = {}
    if prev is not None:
        ins.append(prev)
        specs.append(pl.BlockSpec((8, D), lambda i: (0, 0)))
        aliases = {6: 0}
    return pl.pallas_call(
        body,
        grid=(NBLK,),
        in_specs=specs,
        out_specs=pl.BlockSpec((BE, D), lambda i: (i + b0, 0)),
        out_shape=jax.ShapeDtypeStruct((E, D), jnp.float32),
        input_output_aliases=aliases,
    )(*ins)


def kernel(x, edge_index, edge_attr, x_pos,
           W_mlp1, b_mlp1, W_mlp2, b_mlp2,
           W_lin1, W_lin2, b_lin2, W_lin3, b_lin3,
           W_e, b_e):
    src = edge_index[0]
    dst = edge_index[1]
    xp128 = jnp.pad(x_pos, ((0, 0), (0, 125)))
    h = _tc_h(x, W_lin1)
    ht = jnp.concatenate([h, xp128], axis=1)
    gs, xpd = _sc_gather_src_dst(ht, xp128, src, dst)
    msg = _tc_msg(edge_attr, gs, xpd,
                  W_mlp1, b_mlp1.reshape(1, NF), W_mlp2, b_mlp2.reshape(1, NF),
                  W_lin2)
    zeros_nd = jnp.zeros((NP, D), jnp.float32)
    dst3 = dst.reshape(E // 40, 40)
    parts = _sc_segsum(msg, dst3, zeros_nd)
    xn = _tc_node(parts[:N], parts[NP:NP + N], x, b_lin2, W_lin3, b_lin3)
    E2 = E // 2
    be2 = b_e.reshape(1, D)
    abA = _sc_gather_xn2(xn, src[:E2], dst[:E2])
    eoA = _tc_edge_half(edge_attr, abA, W_e[:D], W_e[D:], be2, 0)
    abB = _sc_gather_xn2(xn, src[E2:], dst[E2:])
    edge_out = _tc_edge_half(edge_attr, abB, W_e[:D], W_e[D:], be2,
                             E2 // 2000, prev=eoA)
    return (xn, edge_out)
